# Initial kernel scaffold; baseline (speedup 1.0000x reference)
import jax
import jax.numpy as jnp
from jax.experimental import pallas as pl


def kernel(seq1, adj, idx_train, idx_test, W_stru, b_stru, W_gat, att_src, att_dst, b_gat, W_a1, b_a1, W_a2, b_a2):
    raise NotImplementedError("write your pallas kernel here")


# speedup vs baseline: 2725.4521x; 2725.4521x over previous
import jax
import jax.numpy as jnp
from jax import lax
from jax.experimental import pallas as pl

N = 1024
NTR = 819
NTE = 205


def _fwd_kernel(seq1_ref, adj_ref, idxtr_ref, idxte_ref, Wstru_ref, bstru_ref,
                Wgat_ref, attsrc_ref, attdst_ref, bgat_ref, Wa1_ref, ba1_ref,
                Wa2_ref, ba2_ref, loss_ref, test_ref):
    f32 = jnp.float32
    seq1 = seq1_ref[...]
    adj = adj_ref[...]

    h = jnp.maximum(
        lax.dot_general(seq1, Wstru_ref[...], (((1,), (1,)), ((), ())),
                        preferred_element_type=f32) + bstru_ref[...], 0.0)
    xw = lax.dot_general(h, Wgat_ref[...], (((1,), (1,)), ((), ())),
                         preferred_element_type=f32)

    a_src = jnp.dot(xw, attsrc_ref[...], preferred_element_type=f32)
    a_dst = lax.dot_general(attdst_ref[...], xw, (((0,), (1,)), ((), ())),
                            preferred_element_type=f32)

    z = a_src + a_dst
    e = jnp.where(z >= 0.0, z, 0.2 * z)

    rows = lax.broadcasted_iota(jnp.int32, (N, N), 0)
    cols = lax.broadcasted_iota(jnp.int32, (N, N), 1)
    cnt = adj + jnp.where(rows == cols, 1.0, 0.0)
    mask = cnt > 0.0

    m = jnp.max(jnp.where(mask, e, -1e30), axis=0, keepdims=True)
    w = cnt * jnp.exp(jnp.where(mask, e - m, -60.0))

    num = lax.dot_general(w, xw, (((0,), (0,)), ((), ())),
                          preferred_element_type=f32)
    ones = jnp.ones((N, 1), f32)
    den = lax.dot_general(w, ones, (((0,), (0,)), ((), ())),
                          preferred_element_type=f32)
    emb = num / (den + 1e-16) + bgat_ref[...]

    x = jnp.maximum(
        lax.dot_general(seq1, Wa1_ref[...], (((1,), (1,)), ((), ())),
                        preferred_element_type=f32) + ba1_ref[...], 0.0)
    x_ = lax.dot_general(x, Wa2_ref[...], (((1,), (1,)), ((), ())),
                         preferred_element_type=f32) + ba2_ref[...]
    da = seq1 - x_
    attr_err = jnp.sqrt(jnp.sum(da * da, axis=1, keepdims=True))

    p = lax.dot_general(emb, emb, (((1,), (1,)), ((), ())),
                        preferred_element_type=f32)
    s = jax.nn.sigmoid(p)
    ds = adj - s
    stru_err = jnp.sqrt(jnp.sum(ds * ds, axis=1, keepdims=True))

    score = 0.5 * attr_err + 0.5 * stru_err

    tr_cols = lax.broadcasted_iota(jnp.int32, (NTR, N), 1)
    oh_tr = (idxtr_ref[...] == tr_cols).astype(f32)
    tr_scores = jnp.dot(oh_tr, score, preferred_element_type=f32)
    loss_ref[...] = jnp.sum(tr_scores, axis=0, keepdims=True) / NTR

    te_cols = lax.broadcasted_iota(jnp.int32, (NTE, N), 1)
    oh_te = (idxte_ref[...] == te_cols).astype(f32)
    test_ref[...] = jnp.dot(oh_te, score, preferred_element_type=f32)


def kernel(seq1, adj, idx_train, idx_test, W_stru, b_stru, W_gat, att_src,
           att_dst, b_gat, W_a1, b_a1, W_a2, b_a2):
    f32 = jnp.float32
    seq1 = jnp.asarray(seq1, f32).reshape(N, 128)
    adj = jnp.asarray(adj, f32).reshape(N, N)
    idxtr = jnp.asarray(idx_train, jnp.int32).reshape(NTR, 1)
    idxte = jnp.asarray(idx_test, jnp.int32).reshape(NTE, 1)

    loss, test = pl.pallas_call(
        _fwd_kernel,
        out_shape=(
            jax.ShapeDtypeStruct((1, 1), f32),
            jax.ShapeDtypeStruct((NTE, 1), f32),
        ),
    )(seq1, adj, idxtr, idxte,
      W_stru, b_stru.reshape(1, 64),
      W_gat, att_src.reshape(128, 1), att_dst.reshape(128, 1),
      b_gat.reshape(1, 128),
      W_a1, b_a1.reshape(1, 64),
      W_a2, b_a2.reshape(1, 128))

    return (loss.reshape(()), test.reshape(NTE))
